# baseline (speedup 1.0000x reference)
import functools

import jax
import jax.numpy as jnp
from jax import lax
from jax.experimental import pallas as pl
from jax.experimental.pallas import tpu as pltpu
from jax.experimental.pallas import tpu_sc as plsc

ROWS, COLS = 32768, 4096
LANES = 16
NUM_CORES = 2
NUM_SUBCORES = 16
NW = NUM_CORES * NUM_SUBCORES
ROWS_PER_W = ROWS // NW
R = 8
CHUNKS = ROWS_PER_W // R
VPR = COLS // LANES
UNROLL = 8


def _rev_body(in_hbm, out_hbm, ibuf, obuf):
    wid = lax.axis_index("s") * NUM_CORES + lax.axis_index("c")
    row0 = wid * ROWS_PER_W

    def chunk(g, carry):
        base = row0 + g * R
        pltpu.sync_copy(in_hbm.at[pl.ds(base, R)], ibuf)
        for r in range(R):
            def inner(kk, c2):
                for u in range(UNROLL):
                    k = kk * UNROLL + u
                    v = ibuf[r, pl.ds(k * LANES, LANES)]
                    obuf[r, pl.ds(COLS - LANES - k * LANES, LANES)] = (
                        lax.rev(v, (0,)))
                return c2
            lax.fori_loop(0, VPR // UNROLL, inner, 0)
        pltpu.sync_copy(obuf, out_hbm.at[pl.ds(base, R)])
        return carry

    lax.fori_loop(0, CHUNKS, chunk, 0)


_rev_kernel = functools.partial(
    pl.kernel,
    out_type=jax.ShapeDtypeStruct((ROWS, COLS), jnp.float32),
    mesh=plsc.VectorSubcoreMesh(
        core_axis_name="c", subcore_axis_name="s",
        num_cores=NUM_CORES, num_subcores=NUM_SUBCORES),
    scratch_types=[
        pltpu.VMEM((R, COLS), jnp.float32),
        pltpu.VMEM((R, COLS), jnp.float32),
    ],
)(_rev_body)


def kernel(inputs):
    return _rev_kernel(inputs)
